# speedup vs baseline: 1.0480x; 1.0480x over previous
import functools

import jax
import jax.numpy as jnp
from jax import lax
from jax.experimental import pallas as pl
from jax.experimental.pallas import tpu as pltpu
from jax.experimental.pallas import tpu_sc as plsc

NUM_ARMS = 1000
NUM_CONTEXTS = 1000
BATCH = 4096
NPRIME = 200

NC = 2
NS = 16
NW = NC * NS
L = 16

N = BATCH * NPRIME
N_PER_W = N // NW
STEPS = N_PER_W // L


def _body(arm_hbm, ctx_hbm, means_hbm, vars_hbm, mean_out, var_out,
          lin_v, ctx_v, gm_v, gv_v, sem_m, sem_v):
  wid = lax.axis_index("s") * NC + lax.axis_index("c")
  base = wid * N_PER_W

  pltpu.sync_copy(arm_hbm.at[pl.ds(base, N_PER_W)], lin_v)
  pltpu.sync_copy(ctx_hbm.at[pl.ds(base, N_PER_W)], ctx_v)

  def step(i, carry):
    sl = pl.ds(i * L, L)
    lin_v[sl] = lin_v[sl] * NUM_CONTEXTS + ctx_v[sl]
    return carry

  lax.fori_loop(0, STEPS, step, 0)

  cm = pltpu.async_copy(means_hbm.at[lin_v], gm_v, sem_m)
  cv = pltpu.async_copy(vars_hbm.at[lin_v], gv_v, sem_v)
  cm.wait()
  cv.wait()

  pltpu.sync_copy(gm_v, mean_out.at[pl.ds(base, N_PER_W)])
  pltpu.sync_copy(gv_v, var_out.at[pl.ds(base, N_PER_W)])


@jax.jit
def _gather_pairs(arm, ctx, means_flat, vars_flat):
  mesh = plsc.VectorSubcoreMesh(core_axis_name="c", subcore_axis_name="s")
  f = pl.kernel(
      _body,
      out_type=(
          jax.ShapeDtypeStruct((N,), jnp.float32),
          jax.ShapeDtypeStruct((N,), jnp.float32),
      ),
      mesh=mesh,
      scratch_types=[
          pltpu.VMEM((N_PER_W,), jnp.int32),
          pltpu.VMEM((N_PER_W,), jnp.int32),
          pltpu.VMEM((N_PER_W,), jnp.float32),
          pltpu.VMEM((N_PER_W,), jnp.float32),
          pltpu.SemaphoreType.DMA,
          pltpu.SemaphoreType.DMA,
      ],
  )
  return f(arm, ctx, means_flat, vars_flat)


def kernel(X, means, vars_):
  Xl = X.astype(jnp.int32)
  arm = Xl[..., 0].reshape(-1)
  ctx = Xl[..., 1].reshape(-1)
  mean_flat, var_flat = _gather_pairs(
      arm, ctx, means.reshape(-1), vars_.reshape(-1))
  return (mean_flat.reshape(BATCH, NPRIME), var_flat.reshape(BATCH, NPRIME))


# speedup vs baseline: 1.1420x; 1.0897x over previous
import functools

import jax
import jax.numpy as jnp
from jax import lax
from jax.experimental import pallas as pl
from jax.experimental.pallas import tpu as pltpu
from jax.experimental.pallas import tpu_sc as plsc

NUM_ARMS = 1000
NUM_CONTEXTS = 1000
BATCH = 4096
NPRIME = 200

NC = 2
NS = 16
NW = NC * NS
L = 16

ROWS_PER_W = BATCH // NW
HALF_ROWS = ROWS_PER_W // 2
N_PER_W = ROWS_PER_W * NPRIME

COL_STEPS = tuple(range(0, NPRIME - L + 1, L))
TAIL_C = NPRIME - L


def _tail_cols(i):
  row = jnp.full((L,), i, jnp.int32)
  col = TAIL_C + lax.iota(jnp.int32, L)
  return row, col


def _body(arm_hbm, ctx_hbm, means_hbm, vars_hbm, mean_out, var_out,
          arm_v, ctx_v, lin_v, gm_v, gv_v, g2d_v, sem_m, sem_v):
  wid = lax.axis_index("s") * NC + lax.axis_index("c")
  row0 = wid * ROWS_PER_W

  for h in range(2):
    pltpu.sync_copy(arm_hbm.at[pl.ds(row0 + h * HALF_ROWS, HALF_ROWS)], arm_v)
    pltpu.sync_copy(ctx_hbm.at[pl.ds(row0 + h * HALF_ROWS, HALF_ROWS)], ctx_v)

    def lin_row(i, carry):
      q0 = (h * HALF_ROWS + i) * NPRIME
      for c in COL_STEPS:
        a = arm_v[i, pl.ds(c, L)]
        b = ctx_v[i, pl.ds(c, L)]
        lin_v[pl.ds(q0 + c, L)] = a * NUM_CONTEXTS + b
      row, col = _tail_cols(i)
      a = plsc.load_gather(arm_v, [row, col])
      b = plsc.load_gather(ctx_v, [row, col])
      lin_v[pl.ds(q0 + TAIL_C, L)] = a * NUM_CONTEXTS + b
      return carry

    lax.fori_loop(0, HALF_ROWS, lin_row, 0)

  cm = pltpu.async_copy(means_hbm.at[lin_v], gm_v, sem_m)
  cv = pltpu.async_copy(vars_hbm.at[lin_v], gv_v, sem_v)

  def unflatten(flat_ref, out_ref):
    for h in range(2):
      def row(i, carry):
        q0 = (h * HALF_ROWS + i) * NPRIME
        for c in COL_STEPS:
          g2d_v[i, pl.ds(c, L)] = flat_ref[pl.ds(q0 + c, L)]
        rr, cc = _tail_cols(i)
        plsc.store_scatter(g2d_v, [rr, cc], flat_ref[pl.ds(q0 + TAIL_C, L)])
        return carry

      lax.fori_loop(0, HALF_ROWS, row, 0)
      pltpu.sync_copy(
          g2d_v, out_ref.at[pl.ds(row0 + h * HALF_ROWS, HALF_ROWS)])

  cm.wait()
  unflatten(gm_v, mean_out)
  cv.wait()
  unflatten(gv_v, var_out)


@jax.jit
def _gather_pairs(arm, ctx, means_flat, vars_flat):
  mesh = plsc.VectorSubcoreMesh(core_axis_name="c", subcore_axis_name="s")
  f = pl.kernel(
      _body,
      out_type=(
          jax.ShapeDtypeStruct((BATCH, NPRIME), jnp.float32),
          jax.ShapeDtypeStruct((BATCH, NPRIME), jnp.float32),
      ),
      mesh=mesh,
      compiler_params=pltpu.CompilerParams(needs_layout_passes=False),
      scratch_types=[
          pltpu.VMEM((HALF_ROWS, NPRIME), jnp.int32),
          pltpu.VMEM((HALF_ROWS, NPRIME), jnp.int32),
          pltpu.VMEM((N_PER_W,), jnp.int32),
          pltpu.VMEM((N_PER_W,), jnp.float32),
          pltpu.VMEM((N_PER_W,), jnp.float32),
          pltpu.VMEM((HALF_ROWS, NPRIME), jnp.float32),
          pltpu.SemaphoreType.DMA,
          pltpu.SemaphoreType.DMA,
      ],
  )
  return f(arm, ctx, means_flat, vars_flat)


def kernel(X, means, vars_):
  Xl = X.astype(jnp.int32)
  arm = Xl[..., 0]
  ctx = Xl[..., 1]
  return _gather_pairs(arm, ctx, means.reshape(-1), vars_.reshape(-1))


# speedup vs baseline: 1.2542x; 1.0983x over previous
import functools

import jax
import jax.numpy as jnp
from jax import lax
from jax.experimental import pallas as pl
from jax.experimental.pallas import tpu as pltpu
from jax.experimental.pallas import tpu_sc as plsc

NUM_ARMS = 1000
NUM_CONTEXTS = 1000
BATCH = 4096
NPRIME = 200

NC = 2
NS = 16
NW = NC * NS
L = 16

ROWS_PER_W = BATCH // NW
N_PER_W = ROWS_PER_W * NPRIME
NBLK = 4
BLK_ROWS = ROWS_PER_W // NBLK
BLK_N = BLK_ROWS * NPRIME

COL_STEPS = tuple(range(0, NPRIME - L + 1, L))
TAIL_C = NPRIME - L


def _body(arm_hbm, ctx_hbm, means_hbm, vars_hbm, mean_out, var_out,
          arm_v, ctx_v, lin_v, gm_v, gv_v, g2d_v, sem_m, sem_v):
  wid = lax.axis_index("s") * NC + lax.axis_index("c")
  row0 = wid * ROWS_PER_W

  for b in range(NBLK):
    r0 = row0 + b * BLK_ROWS
    pltpu.sync_copy(arm_hbm.at[pl.ds(r0, BLK_ROWS)], arm_v)
    pltpu.sync_copy(ctx_hbm.at[pl.ds(r0, BLK_ROWS)], ctx_v)

    def lin_row(i, carry):
      q0 = (b * BLK_ROWS + i) * NPRIME
      for c in COL_STEPS:
        a = arm_v[i, pl.ds(c, L)]
        x = ctx_v[i, pl.ds(c, L)]
        lin_v[pl.ds(q0 + c, L)] = a * NUM_CONTEXTS + x
      row = jnp.full((L,), i, jnp.int32)
      col = TAIL_C + lax.iota(jnp.int32, L)
      a = plsc.load_gather(arm_v, [row, col])
      x = plsc.load_gather(ctx_v, [row, col])
      lin_v[pl.ds(q0 + TAIL_C, L)] = a * NUM_CONTEXTS + x
      return carry

    lax.fori_loop(0, BLK_ROWS, lin_row, 0)
    idx = lin_v.at[pl.ds(b * BLK_N, BLK_N)]
    pltpu.async_copy(means_hbm.at[idx], gm_v.at[pl.ds(b * BLK_N, BLK_N)],
                     sem_m)
    pltpu.async_copy(vars_hbm.at[idx], gv_v.at[pl.ds(b * BLK_N, BLK_N)],
                     sem_v)

  def unflatten_block(b, flat_ref, out_ref):
    def row(i, carry):
      q0 = (b * BLK_ROWS + i) * NPRIME
      for c in COL_STEPS:
        g2d_v[i, pl.ds(c, L)] = flat_ref[pl.ds(q0 + c, L)]
      rr = jnp.full((L,), i, jnp.int32)
      cc = TAIL_C + lax.iota(jnp.int32, L)
      plsc.store_scatter(g2d_v, [rr, cc], flat_ref[pl.ds(q0 + TAIL_C, L)])
      return carry

    lax.fori_loop(0, BLK_ROWS, row, 0)
    pltpu.sync_copy(g2d_v, out_ref.at[pl.ds(row0 + b * BLK_ROWS, BLK_ROWS)])

  for b in range(NBLK):
    pltpu.make_async_copy(means_hbm.at[pl.ds(0, BLK_N)],
                          gm_v.at[pl.ds(b * BLK_N, BLK_N)], sem_m).wait()
    unflatten_block(b, gm_v, mean_out)
    pltpu.make_async_copy(means_hbm.at[pl.ds(0, BLK_N)],
                          gv_v.at[pl.ds(b * BLK_N, BLK_N)], sem_v).wait()
    unflatten_block(b, gv_v, var_out)


@jax.jit
def _gather_pairs(arm, ctx, means_flat, vars_flat):
  mesh = plsc.VectorSubcoreMesh(core_axis_name="c", subcore_axis_name="s")
  f = pl.kernel(
      _body,
      out_type=(
          jax.ShapeDtypeStruct((BATCH, NPRIME), jnp.float32),
          jax.ShapeDtypeStruct((BATCH, NPRIME), jnp.float32),
      ),
      mesh=mesh,
      compiler_params=pltpu.CompilerParams(needs_layout_passes=False),
      scratch_types=[
          pltpu.VMEM((BLK_ROWS, NPRIME), jnp.int32),
          pltpu.VMEM((BLK_ROWS, NPRIME), jnp.int32),
          pltpu.VMEM((N_PER_W,), jnp.int32),
          pltpu.VMEM((N_PER_W,), jnp.float32),
          pltpu.VMEM((N_PER_W,), jnp.float32),
          pltpu.VMEM((BLK_ROWS, NPRIME), jnp.float32),
          pltpu.SemaphoreType.DMA,
          pltpu.SemaphoreType.DMA,
      ],
  )
  return f(arm, ctx, means_flat, vars_flat)


def kernel(X, means, vars_):
  Xl = X.astype(jnp.int32)
  arm = Xl[..., 0]
  ctx = Xl[..., 1]
  return _gather_pairs(arm, ctx, means.reshape(-1), vars_.reshape(-1))
